# Initial kernel scaffold; baseline (speedup 1.0000x reference)
import jax
import jax.numpy as jnp
from jax.experimental import pallas as pl


def kernel(x, edge_index, W_in, b_in, W_h0, b_h0, W_h1, b_h1, W_out, b_out):
    raise NotImplementedError("write your pallas kernel here")


# speedup vs baseline: 2.3938x; 2.3938x over previous
import functools

import jax
import jax.numpy as jnp
from jax import lax
from jax.experimental import pallas as pl
from jax.experimental.pallas import tpu as pltpu
from jax.experimental.pallas import tpu_sc as plsc

N = 10000
D = 128
NC = 2
NT = 16
CHUNK = 128
R_PAD = 10240
ROWS_T = R_PAD // NT
RB = 400
GRID = N // RB

_sc_mesh = plsc.VectorSubcoreMesh(core_axis_name="c", subcore_axis_name="s")


def _deg_call(dst_flat, e_pad):
    e_w = e_pad // (NC * NT)

    @functools.partial(
        pl.kernel,
        out_type=jax.ShapeDtypeStruct((NC * R_PAD,), jnp.float32),
        mesh=_sc_mesh,
        scratch_types=[
            pltpu.VMEM((e_w,), jnp.int32),
            pltpu.VMEM((R_PAD,), jnp.float32),
            pltpu.VMEM((ROWS_T,), jnp.float32),
            pltpu.VMEM((ROWS_T,), jnp.float32),
            pltpu.VMEM_SHARED((NT, R_PAD), jnp.float32),
        ],
        compiler_params=pltpu.CompilerParams(needs_layout_passes=False),
    )
    def deg_kernel(dst_hbm, out_hbm, dstv, hist, buf, acc, shared):
        c = lax.axis_index("c")
        t = lax.axis_index("s")
        pltpu.sync_copy(dst_hbm.at[pl.ds((c * NT + t) * e_w, e_w)], dstv)
        zero16 = jnp.zeros((16,), jnp.float32)
        ones16 = jnp.ones((16,), jnp.float32)

        def zbody(i, carry):
            hist[pl.ds(i * 16, 16)] = zero16
            return carry

        lax.fori_loop(0, R_PAD // 16, zbody, 0)

        def scat(i, carry):
            idx = dstv[pl.ds(i * 16, 16)]
            plsc.addupdate_scatter(hist, [idx], ones16)
            return carry

        lax.fori_loop(0, e_w // 16, scat, 0)
        pltpu.sync_copy(hist, shared.at[t])
        plsc.subcore_barrier()

        def z2(i, carry):
            acc[pl.ds(i * 16, 16)] = zero16
            return carry

        lax.fori_loop(0, ROWS_T // 16, z2, 0)
        for r in range(NT):
            pltpu.sync_copy(shared.at[r, pl.ds(t * ROWS_T, ROWS_T)], buf)

            def addb(i, carry):
                acc[pl.ds(i * 16, 16)] = acc[pl.ds(i * 16, 16)] + buf[pl.ds(i * 16, 16)]
                return carry

            lax.fori_loop(0, ROWS_T // 16, addb, 0)
        pltpu.sync_copy(acc, out_hbm.at[pl.ds(c * R_PAD + t * ROWS_T, ROWS_T)])

    return deg_kernel(dst_flat)


def _propagate(gtab, src_idx, dst_idx, zeros_blk, c_w):
    @functools.partial(
        pl.kernel,
        out_type=jax.ShapeDtypeStruct((NC * R_PAD, D), jnp.float32),
        mesh=_sc_mesh,
        scratch_types=[
            pltpu.VMEM((c_w, CHUNK), jnp.int32),
            pltpu.VMEM((c_w, CHUNK), jnp.int32),
            pltpu.VMEM((CHUNK, D), jnp.float32),
            pltpu.VMEM_SHARED((R_PAD, D), jnp.float32),
            pltpu.SemaphoreType.DMA,
        ],
        compiler_params=pltpu.CompilerParams(needs_layout_passes=False),
    )
    def prop_kernel(gtab_hbm, src_hbm, dst_hbm, z_hbm, out_hbm,
                    src_v, dst_v, rows_a, accum, sem_a):
        c = lax.axis_index("c")
        t = lax.axis_index("s")
        base = t * ROWS_T
        pltpu.sync_copy(z_hbm, accum.at[pl.ds(base, ROWS_T)])
        pltpu.sync_copy(src_hbm.at[pl.ds((c * NT + t) * c_w, c_w)], src_v)
        pltpu.sync_copy(dst_hbm.at[pl.ds((c * NT + t) * c_w, c_w)], dst_v)
        plsc.subcore_barrier()

        def step(j, carry):
            pltpu.async_copy(gtab_hbm.at[src_v.at[j]], rows_a, sem_a).wait()
            pltpu.sync_copy(rows_a, accum.at[dst_v.at[j]], add=True)
            return carry

        lax.fori_loop(0, c_w, step, 0)
        plsc.subcore_barrier()
        pltpu.sync_copy(accum.at[pl.ds(base, ROWS_T)],
                        out_hbm.at[pl.ds(c * R_PAD + base, ROWS_T)])

    return prop_kernel(gtab, src_idx, dst_idx, zeros_blk)


def _tc_first(x, W, deg0, deg1):
    def body(x_ref, w_ref, d0_ref, d1_ref, g_ref, dinv_ref):
        dinv = lax.rsqrt(jnp.maximum(d0_ref[...] + d1_ref[...], 1.0))
        m = jnp.dot(x_ref[...], w_ref[...], preferred_element_type=jnp.float32)
        g_ref[...] = m * dinv
        dinv_ref[...] = dinv

    return pl.pallas_call(
        body,
        grid=(GRID,),
        in_specs=[
            pl.BlockSpec((RB, D), lambda i: (i, 0)),
            pl.BlockSpec((D, D), lambda i: (0, 0)),
            pl.BlockSpec((RB, 1), lambda i: (i, 0)),
            pl.BlockSpec((RB, 1), lambda i: (i, 0)),
        ],
        out_specs=[
            pl.BlockSpec((RB, D), lambda i: (i, 0)),
            pl.BlockSpec((RB, 1), lambda i: (i, 0)),
        ],
        out_shape=[
            jax.ShapeDtypeStruct((N, D), jnp.float32),
            jax.ShapeDtypeStruct((N, 1), jnp.float32),
        ],
    )(x, W, deg0, deg1)


def _tc_mid(s, dinv, b, res, W):
    has_res = res is not None

    def body(*refs):
        if has_res:
            s_ref, dinv_ref, b_ref, res_ref, w_ref, h_ref, g_ref = refs
        else:
            s_ref, dinv_ref, b_ref, w_ref, h_ref, g_ref = refs
        dv = dinv_ref[...]
        h = (s_ref[0] + s_ref[1]) * dv + b_ref[...]
        if has_res:
            h = h + res_ref[...]
        h = jnp.maximum(h, 0.0)
        h_ref[...] = h
        g_ref[...] = jnp.dot(h, w_ref[...], preferred_element_type=jnp.float32) * dv

    in_specs = [
        pl.BlockSpec((NC, RB, D), lambda i: (0, i, 0)),
        pl.BlockSpec((RB, 1), lambda i: (i, 0)),
        pl.BlockSpec((1, D), lambda i: (0, 0)),
    ]
    args = [s, dinv, b]
    if has_res:
        in_specs.append(pl.BlockSpec((RB, D), lambda i: (i, 0)))
        args.append(res)
    in_specs.append(pl.BlockSpec((D, D), lambda i: (0, 0)))
    args.append(W)
    return pl.pallas_call(
        body,
        grid=(GRID,),
        in_specs=in_specs,
        out_specs=[
            pl.BlockSpec((RB, D), lambda i: (i, 0)),
            pl.BlockSpec((RB, D), lambda i: (i, 0)),
        ],
        out_shape=[
            jax.ShapeDtypeStruct((N, D), jnp.float32),
            jax.ShapeDtypeStruct((N, D), jnp.float32),
        ],
    )(*args)


def _tc_last(s, dinv, b):
    def body(s_ref, dinv_ref, b_ref, out_ref):
        out_ref[...] = (s_ref[0] + s_ref[1]) * dinv_ref[...] + b_ref[...]

    return pl.pallas_call(
        body,
        grid=(GRID,),
        in_specs=[
            pl.BlockSpec((NC, RB, D), lambda i: (0, i, 0)),
            pl.BlockSpec((RB, 1), lambda i: (i, 0)),
            pl.BlockSpec((1, D), lambda i: (0, 0)),
        ],
        out_specs=pl.BlockSpec((RB, D), lambda i: (i, 0)),
        out_shape=jax.ShapeDtypeStruct((N, D), jnp.float32),
    )(s, dinv, b)


def kernel(x, edge_index, W_in, b_in, W_h0, b_h0, W_h1, b_h1, W_out, b_out):
    src = edge_index[0]
    dst = edge_index[1]
    e_tot = src.shape[0] + N
    c_w = -(-e_tot // (NC * NT * CHUNK))
    c_w = -(-c_w // 8) * 8
    e_pad = NC * NT * c_w * CHUNK
    loop_idx = jnp.arange(N, dtype=jnp.int32)
    pad = e_pad - e_tot
    src_f = jnp.concatenate([src, loop_idx, jnp.zeros((pad,), jnp.int32)])
    dst_f = jnp.concatenate([dst, loop_idx, jnp.full((pad,), N, jnp.int32)])
    src_idx = src_f.reshape(NC * NT * c_w, CHUNK)
    dst_idx = dst_f.reshape(NC * NT * c_w, CHUNK)
    zeros_blk = jnp.zeros((ROWS_T, D), jnp.float32)

    deg = _deg_call(dst_f, e_pad).reshape(NC, R_PAD, 1)
    g1, dinv = _tc_first(x, W_in, deg[0], deg[1])
    s1 = _propagate(g1, src_idx, dst_idx, zeros_blk, c_w)
    h1, g2 = _tc_mid(s1.reshape(NC, R_PAD, D), dinv, b_in.reshape(1, D), None, W_h0)
    s2 = _propagate(g2, src_idx, dst_idx, zeros_blk, c_w)
    h2, g3 = _tc_mid(s2.reshape(NC, R_PAD, D), dinv, b_h0.reshape(1, D), h1, W_h1)
    s3 = _propagate(g3, src_idx, dst_idx, zeros_blk, c_w)
    _, g4 = _tc_mid(s3.reshape(NC, R_PAD, D), dinv, b_h1.reshape(1, D), h2, W_out)
    s4 = _propagate(g4, src_idx, dst_idx, zeros_blk, c_w)
    return _tc_last(s4.reshape(NC, R_PAD, D), dinv, b_out.reshape(1, D))
